# baseline (speedup 1.0000x reference)
import dataclasses
import functools

import jax
import jax.numpy as jnp
from jax import lax
from jax.experimental import pallas as pl
from jax.experimental.pallas import tpu as pltpu
from jax.experimental.pallas import tpu_sc as plsc

N_NODES = 10000
N_PAD = 10240
INP_DIM = 128
HID_DIM = 256
OUT_DIM = 128
APPNP_K = 10
ALPHA = 0.1

NC = 2
NS = 16
CHUNK = 256
GRP = 11
NGRP = 4
NCHUNK = 128
CHUNKS_PT = 176
EPT = CHUNKS_PT * NCHUNK
E_PAD = EPT * NS
CHUNKS_PT2 = GRP * NGRP
NODES_PT = N_PAD // NS
UPD = 128

_mesh = plsc.VectorSubcoreMesh(core_axis_name="c", subcore_axis_name="s")

_sc_params = pltpu.CompilerParams()
if "needs_layout_passes" in pltpu.CompilerParams.__dataclass_fields__:
    _sc_params = dataclasses.replace(_sc_params, needs_layout_passes=False)


def _mlp_body(x_ref, w1t_ref, b1_ref, w2t_ref, b2_ref, out_ref):
    h = jnp.dot(x_ref[...], w1t_ref[...], preferred_element_type=jnp.float32)
    h = h + b1_ref[...]
    h = jnp.where(h > 0, h, jnp.exp(jnp.minimum(h, 0.0)) - 1.0)
    out = jnp.dot(h, w2t_ref[...], preferred_element_type=jnp.float32)
    out_ref[...] = out + b2_ref[...]


def _mlp(x, W1, b1, W2, b2):
    n = x.shape[0]
    return pl.pallas_call(
        _mlp_body,
        out_shape=jax.ShapeDtypeStruct((n, OUT_DIM), jnp.float32),
    )(x, W1.T, b1[None, :], W2.T, b2[None, :])


def _rsqrt16(d):
    i = plsc.bitcast(d, jnp.int32)
    i = jnp.int32(0x5F3759DF) - lax.shift_right_logical(i, 1)
    y = plsc.bitcast(i, jnp.float32)
    for _ in range(3):
        y = y * (1.5 - 0.5 * d * y * y)
    return y


def _norm_body(src_hbm, dst_hbm, w_hbm, norm_hbm,
               srcv, dstv, wv, dinv_v, deg_sh):
    c = lax.axis_index("c")
    s = lax.axis_index("s")

    @pl.when(c == 0)
    def _():
        npt = N_PAD // NS

        @pl.loop(0, npt // 16)
        def _(i):
            dinv_v[pl.ds(i * 16, 16)] = jnp.zeros((16,), jnp.float32)

        pltpu.sync_copy(dinv_v.at[pl.ds(0, npt)],
                        deg_sh.at[pl.ds(s * npt, npt)])
        pltpu.sync_copy(src_hbm.at[s], srcv)
        pltpu.sync_copy(dst_hbm.at[s], dstv)
        pltpu.sync_copy(w_hbm.at[s], wv)
        plsc.subcore_barrier()

        @pl.loop(0, CHUNKS_PT)
        def _(j):
            pltpu.sync_copy(wv.at[j], deg_sh.at[dstv.at[j]], add=True)

        plsc.subcore_barrier()

        pltpu.sync_copy(deg_sh, dinv_v)

        @pl.loop(0, N_PAD // 16)
        def _(i):
            d = dinv_v[pl.ds(i * 16, 16)]
            y = _rsqrt16(jnp.maximum(d, 1e-12))
            dinv_v[pl.ds(i * 16, 16)] = jnp.where(d > 0, y, 0.0)

        @pl.loop(0, CHUNKS_PT)
        def _(j):
            @pl.loop(0, NCHUNK // 16)
            def _(e):
                sl = pl.ds(e * 16, 16)
                a = plsc.load_gather(dinv_v, [srcv[j, sl]])
                b = plsc.load_gather(dinv_v, [dstv[j, sl]])
                wv[j, sl] = a * wv[j, sl] * b

        pltpu.sync_copy(wv, norm_hbm.at[s])


def _edge_norm(src3, dst3, w3):
    kfn = pl.kernel(
        _norm_body,
        out_type=jax.ShapeDtypeStruct((NS, CHUNKS_PT, NCHUNK), jnp.float32),
        mesh=_mesh,
        scratch_types=[
            pltpu.VMEM((CHUNKS_PT, NCHUNK), jnp.int32),
            pltpu.VMEM((CHUNKS_PT, NCHUNK), jnp.int32),
            pltpu.VMEM((CHUNKS_PT, NCHUNK), jnp.float32),
            pltpu.VMEM((N_PAD,), jnp.float32),
            pltpu.VMEM_SHARED((N_PAD,), jnp.float32),
        ],
        compiler_params=_sc_params,
    )
    return kfn(src3, dst3, w3)


def _prop_body(x_hbm, src_hbm, dst_hbm, norm_hbm, agg_hbm,
               src_v, dst_v, norm_v, src_cur, dst_cur, rows0, agg_sh):
    c = lax.axis_index("c")
    s = lax.axis_index("s")
    w = c * NS + s

    @pl.loop(0, UPD)
    def _(r):
        for f in range(OUT_DIM // 16):
            rows0[r, pl.ds(f * 16, 16)] = jnp.zeros((16,), jnp.float32)

    for k in range(NODES_PT // UPD):
        pltpu.sync_copy(rows0.at[pl.ds(0, UPD)],
                        agg_sh.at[pl.ds(s * NODES_PT + k * UPD, UPD)])
    plsc.subcore_barrier()

    @pl.loop(0, NGRP)
    def _(q):
        pltpu.sync_copy(src_hbm.at[w * NGRP + q], src_v)
        pltpu.sync_copy(dst_hbm.at[w * NGRP + q], dst_v)
        pltpu.sync_copy(norm_hbm.at[w * NGRP + q], norm_v)

        @pl.loop(0, GRP)
        def _(u):
            @pl.loop(0, CHUNK // 16)
            def _(g):
                sl = pl.ds(g * 16, 16)
                sl_src = pl.ds(u * CHUNK + g * 16, 16)
                src_cur[sl] = src_v[sl_src]
                dst_cur[sl] = dst_v[sl_src]

            pltpu.sync_copy(x_hbm.at[src_cur], rows0)

            @pl.loop(0, CHUNK // 16)
            def _(g):
                nv = norm_v[pl.ds(u * CHUNK + g * 16, 16)]
                for i in range(16):
                    n = nv[i]
                    e = g * 16 + i
                    for f in range(OUT_DIM // 16):
                        sl = pl.ds(f * 16, 16)
                        rows0[e, sl] = rows0[e, sl] * n

            pltpu.sync_copy(rows0, agg_sh.at[dst_cur], add=True)

    plsc.subcore_barrier()

    for k in range(NODES_PT // UPD):
        nsl = pl.ds(s * NODES_PT + k * UPD, UPD)
        pltpu.sync_copy(agg_sh.at[nsl], rows0.at[pl.ds(0, UPD)])
        pltpu.sync_copy(rows0.at[pl.ds(0, UPD)], agg_hbm.at[c].at[nsl])


def _propagate(x, src2, dst2, norm2):
    kfn = pl.kernel(
        _prop_body,
        out_type=jax.ShapeDtypeStruct((NC, N_PAD, OUT_DIM), jnp.float32),
        mesh=_mesh,
        scratch_types=[
            pltpu.VMEM((GRP * CHUNK,), jnp.int32),
            pltpu.VMEM((GRP * CHUNK,), jnp.int32),
            pltpu.VMEM((GRP * CHUNK,), jnp.float32),
            pltpu.VMEM((CHUNK,), jnp.int32),
            pltpu.VMEM((CHUNK,), jnp.int32),
            pltpu.VMEM((CHUNK, OUT_DIM), jnp.float32),
            pltpu.VMEM_SHARED((N_PAD, OUT_DIM), jnp.float32),
        ],
        compiler_params=_sc_params,
    )
    return kfn(x, src2, dst2, norm2)


def _update_body(a0_ref, a1_ref, h_ref, out_ref):
    out_ref[...] = (1.0 - ALPHA) * (a0_ref[...] + a1_ref[...]) \
        + ALPHA * h_ref[...]


def _update(agg2, h):
    return pl.pallas_call(
        _update_body,
        out_shape=jax.ShapeDtypeStruct((N_PAD, OUT_DIM), jnp.float32),
    )(agg2[0], agg2[1], h)


def kernel(x, edge_index, edge_attr, W1, b1, W2, b2):
    src = edge_index[0]
    dst = edge_index[1]
    loop = jnp.arange(N_NODES, dtype=src.dtype)
    pad = E_PAD - (src.shape[0] + N_NODES)
    zpad_i = jnp.zeros((pad,), src.dtype)
    zpad_f = jnp.zeros((pad,), jnp.float32)
    src_flat = jnp.concatenate([src, loop, zpad_i])
    dst_flat = jnp.concatenate([dst, loop, zpad_i])
    w_flat = jnp.concatenate(
        [edge_attr, jnp.ones((N_NODES,), jnp.float32), zpad_f])
    src3 = src_flat.reshape(NS, CHUNKS_PT, NCHUNK)
    dst3 = dst_flat.reshape(NS, CHUNKS_PT, NCHUNK)
    w3 = w_flat.reshape(NS, CHUNKS_PT, NCHUNK)

    norm3 = _edge_norm(src3, dst3, w3)
    ntg = NC * NS * NGRP
    src2 = src_flat.reshape(ntg, GRP * CHUNK)
    dst2 = dst_flat.reshape(ntg, GRP * CHUNK)
    norm2 = norm3.reshape(ntg, GRP * CHUNK)

    h0 = _mlp(x, W1, b1, W2, b2)
    h0 = jnp.concatenate(
        [h0, jnp.zeros((N_PAD - N_NODES, OUT_DIM), jnp.float32)])

    xcur = h0
    for _ in range(APPNP_K):
        xcur = _update(_propagate(xcur, src2, dst2, norm2), h0)
    h1 = xcur
    for _ in range(APPNP_K):
        xcur = _update(_propagate(xcur, src2, dst2, norm2), h1)

    return xcur[:N_NODES]


# speedup vs baseline: 1.7538x; 1.7538x over previous
import dataclasses
import functools

import jax
import jax.numpy as jnp
from jax import lax
from jax.experimental import pallas as pl
from jax.experimental.pallas import tpu as pltpu
from jax.experimental.pallas import tpu_sc as plsc

N_NODES = 10000
N_PAD = 10240
INP_DIM = 128
HID_DIM = 256
OUT_DIM = 128
APPNP_K = 10
ALPHA = 0.1

NC = 2
NS = 16
CHUNK = 128
CHUNKS_PT = 168
EPT = CHUNKS_PT * CHUNK
E_PAD = EPT * NS
CHUNKS_PT2 = CHUNKS_PT // 2
NODES_PT = N_PAD // NS
UPD = 128

_mesh = plsc.VectorSubcoreMesh(core_axis_name="c", subcore_axis_name="s")

_sc_params = pltpu.CompilerParams()
if "needs_layout_passes" in pltpu.CompilerParams.__dataclass_fields__:
    _sc_params = dataclasses.replace(_sc_params, needs_layout_passes=False)


def _mlp_body(x_ref, w1t_ref, b1_ref, w2t_ref, b2_ref, out_ref):
    h = jnp.dot(x_ref[...], w1t_ref[...], preferred_element_type=jnp.float32)
    h = h + b1_ref[...]
    h = jnp.where(h > 0, h, jnp.exp(jnp.minimum(h, 0.0)) - 1.0)
    out = jnp.dot(h, w2t_ref[...], preferred_element_type=jnp.float32)
    out_ref[...] = out + b2_ref[...]


def _mlp(x, W1, b1, W2, b2):
    n = x.shape[0]
    return pl.pallas_call(
        _mlp_body,
        out_shape=jax.ShapeDtypeStruct((n, OUT_DIM), jnp.float32),
    )(x, W1.T, b1[None, :], W2.T, b2[None, :])


def _rsqrt16(d):
    i = plsc.bitcast(d, jnp.int32)
    i = jnp.int32(0x5F3759DF) - lax.shift_right_logical(i, 1)
    y = plsc.bitcast(i, jnp.float32)
    for _ in range(3):
        y = y * (1.5 - 0.5 * d * y * y)
    return y


def _norm_body(src_hbm, dst_hbm, w_hbm, norm_hbm,
               srcv, dstv, wv, dinv_v, deg_sh):
    c = lax.axis_index("c")
    s = lax.axis_index("s")

    @pl.when(c == 0)
    def _():
        npt = N_PAD // NS

        @pl.loop(0, npt // 16)
        def _(i):
            dinv_v[pl.ds(i * 16, 16)] = jnp.zeros((16,), jnp.float32)

        pltpu.sync_copy(dinv_v.at[pl.ds(0, npt)],
                        deg_sh.at[pl.ds(s * npt, npt)])
        pltpu.sync_copy(src_hbm.at[s], srcv)
        pltpu.sync_copy(dst_hbm.at[s], dstv)
        pltpu.sync_copy(w_hbm.at[s], wv)
        plsc.subcore_barrier()

        @pl.loop(0, CHUNKS_PT)
        def _(j):
            pltpu.sync_copy(wv.at[j], deg_sh.at[dstv.at[j]], add=True)

        plsc.subcore_barrier()

        pltpu.sync_copy(deg_sh, dinv_v)

        @pl.loop(0, N_PAD // 16)
        def _(i):
            d = dinv_v[pl.ds(i * 16, 16)]
            y = _rsqrt16(jnp.maximum(d, 1e-12))
            dinv_v[pl.ds(i * 16, 16)] = jnp.where(d > 0, y, 0.0)

        @pl.loop(0, CHUNKS_PT)
        def _(j):
            @pl.loop(0, CHUNK // 16)
            def _(e):
                sl = pl.ds(e * 16, 16)
                a = plsc.load_gather(dinv_v, [srcv[j, sl]])
                b = plsc.load_gather(dinv_v, [dstv[j, sl]])
                wv[j, sl] = a * wv[j, sl] * b

        pltpu.sync_copy(wv, norm_hbm.at[s])


def _edge_norm(src3, dst3, w3):
    kfn = pl.kernel(
        _norm_body,
        out_type=jax.ShapeDtypeStruct((NS, CHUNKS_PT, CHUNK), jnp.float32),
        mesh=_mesh,
        scratch_types=[
            pltpu.VMEM((CHUNKS_PT, CHUNK), jnp.int32),
            pltpu.VMEM((CHUNKS_PT, CHUNK), jnp.int32),
            pltpu.VMEM((CHUNKS_PT, CHUNK), jnp.float32),
            pltpu.VMEM((N_PAD,), jnp.float32),
            pltpu.VMEM_SHARED((N_PAD,), jnp.float32),
        ],
        compiler_params=_sc_params,
    )
    return kfn(src3, dst3, w3)


def _prop_body(x_hbm, meta_hbm, agg_hbm,
               m0, m1, m2, m3, r0, r1, sems, agg_sh):
    c = lax.axis_index("c")
    s = lax.axis_index("s")
    w = c * NS + s
    base = w * CHUNKS_PT2
    metas = (m0, m1, m2, m3)
    rows = (r0, r1)

    @pl.loop(0, UPD)
    def _(r):
        for f in range(OUT_DIM // 16):
            r0[r, pl.ds(f * 16, 16)] = jnp.zeros((16,), jnp.float32)

    for k in range(NODES_PT // UPD):
        pltpu.sync_copy(r0, agg_sh.at[pl.ds(s * NODES_PT + k * UPD, UPD)])
    plsc.subcore_barrier()

    def start_meta(m, j):
        pltpu.async_copy(meta_hbm.at[base + j], metas[m], sems.at[4 + m])

    def wait_meta(m):
        pltpu.make_async_copy(meta_hbm.at[base], metas[m],
                              sems.at[4 + m]).wait()

    def start_gather(r, m):
        pltpu.async_copy(x_hbm.at[metas[m].at[0]], rows[r], sems.at[r])

    def wait_gather(r):
        pltpu.make_async_copy(x_hbm.at[metas[0].at[0]], rows[r],
                              sems.at[r]).wait()

    def start_scatter(r, m):
        pltpu.async_copy(rows[r], agg_sh.at[metas[m].at[1]], sems.at[2 + r],
                         add=True)

    def wait_scatter(r):
        pltpu.make_async_copy(rows[r], agg_sh.at[metas[0].at[1]],
                              sems.at[2 + r]).wait()

    def scale(r, m):
        @pl.loop(0, CHUNK // 16)
        def _(g):
            nv = plsc.bitcast(metas[m][2, pl.ds(g * 16, 16)], jnp.float32)
            for i in range(16):
                n = nv[i]
                e = g * 16 + i
                for f in range(OUT_DIM // 16):
                    sl = pl.ds(f * 16, 16)
                    rows[r][e, sl] = rows[r][e, sl] * n

    start_meta(0, 0)
    start_meta(1, 1)
    start_meta(2, 2)
    wait_meta(0)
    start_gather(0, 0)

    @pl.loop(0, CHUNKS_PT2, step=4)
    def _(j):
        for b in range(4):
            cc = j + b
            r = b % 2
            m = b % 4
            wait_gather(r)
            scale(r, m)
            start_scatter(r, m)
            if b == 0:
                @pl.when(j > 0)
                def _():
                    wait_scatter(1 - r)
            else:
                wait_scatter(1 - r)

            @pl.when(cc + 3 < CHUNKS_PT2)
            def _():
                start_meta((m + 3) % 4, cc + 3)

            @pl.when(cc + 1 < CHUNKS_PT2)
            def _():
                wait_meta((m + 1) % 4)
                start_gather(1 - r, (m + 1) % 4)

    wait_scatter((CHUNKS_PT2 - 1) % 2)

    plsc.subcore_barrier()

    for k in range(NODES_PT // UPD):
        nsl = pl.ds(s * NODES_PT + k * UPD, UPD)
        pltpu.sync_copy(agg_sh.at[nsl], r0)
        pltpu.sync_copy(r0, agg_hbm.at[c].at[nsl])


def _propagate(x, meta):
    kfn = pl.kernel(
        _prop_body,
        out_type=jax.ShapeDtypeStruct((NC, N_PAD, OUT_DIM), jnp.float32),
        mesh=_mesh,
        scratch_types=(
            [pltpu.VMEM((3, CHUNK), jnp.int32)] * 4
            + [pltpu.VMEM((CHUNK, OUT_DIM), jnp.float32)] * 2
            + [pltpu.SemaphoreType.DMA((8,)),
               pltpu.VMEM_SHARED((N_PAD, OUT_DIM), jnp.float32)]
        ),
        compiler_params=_sc_params,
    )
    return kfn(x, meta)


def _update_body(a0_ref, a1_ref, h_ref, out_ref):
    out_ref[...] = (1.0 - ALPHA) * (a0_ref[...] + a1_ref[...]) \
        + ALPHA * h_ref[...]


def _update(agg2, h):
    return pl.pallas_call(
        _update_body,
        out_shape=jax.ShapeDtypeStruct((N_PAD, OUT_DIM), jnp.float32),
    )(agg2[0], agg2[1], h)


def kernel(x, edge_index, edge_attr, W1, b1, W2, b2):
    src = edge_index[0]
    dst = edge_index[1]
    loop = jnp.arange(N_NODES, dtype=src.dtype)
    pad = E_PAD - (src.shape[0] + N_NODES)
    zpad_i = jnp.zeros((pad,), src.dtype)
    zpad_f = jnp.zeros((pad,), jnp.float32)
    src_flat = jnp.concatenate([src, loop, zpad_i])
    dst_flat = jnp.concatenate([dst, loop, zpad_i])
    w_flat = jnp.concatenate(
        [edge_attr, jnp.ones((N_NODES,), jnp.float32), zpad_f])
    src3 = src_flat.reshape(NS, CHUNKS_PT, CHUNK)
    dst3 = dst_flat.reshape(NS, CHUNKS_PT, CHUNK)
    w3 = w_flat.reshape(NS, CHUNKS_PT, CHUNK)

    norm3 = _edge_norm(src3, dst3, w3)
    norm_bits = lax.bitcast_convert_type(norm3.reshape(-1), jnp.int32)
    nchunks = E_PAD // CHUNK
    meta = jnp.stack(
        [src_flat.reshape(nchunks, CHUNK),
         dst_flat.reshape(nchunks, CHUNK),
         norm_bits.reshape(nchunks, CHUNK)], axis=1)

    h0 = _mlp(x, W1, b1, W2, b2)
    h0 = jnp.concatenate(
        [h0, jnp.zeros((N_PAD - N_NODES, OUT_DIM), jnp.float32)])

    xcur = h0
    for _ in range(APPNP_K):
        xcur = _update(_propagate(xcur, meta), h0)
    h1 = xcur
    for _ in range(APPNP_K):
        xcur = _update(_propagate(xcur, meta), h1)

    return xcur[:N_NODES]


# speedup vs baseline: 3.8144x; 2.1749x over previous
import dataclasses
import functools

import jax
import jax.numpy as jnp
from jax import lax
from jax.experimental import pallas as pl
from jax.experimental.pallas import tpu as pltpu
from jax.experimental.pallas import tpu_sc as plsc

N_NODES = 10000
N_PAD = 10240
INP_DIM = 128
HID_DIM = 256
OUT_DIM = 128
APPNP_K = 10
ALPHA = 0.1

NC = 2
NS = 16
CHUNK = 128
CHUNKS_PT = 162
EPT = CHUNKS_PT * CHUNK
E_PAD = EPT * NS
CHUNKS_PT2 = CHUNKS_PT // 2
NODES_PT = N_PAD // NS
UPD = 128

_mesh = plsc.VectorSubcoreMesh(core_axis_name="c", subcore_axis_name="s")

_sc_params = pltpu.CompilerParams()
if "needs_layout_passes" in pltpu.CompilerParams.__dataclass_fields__:
    _sc_params = dataclasses.replace(_sc_params, needs_layout_passes=False)


def _mlp_body(x_ref, w1t_ref, b1_ref, w2t_ref, b2_ref, out_ref):
    h = jnp.dot(x_ref[...], w1t_ref[...], preferred_element_type=jnp.float32)
    h = h + b1_ref[...]
    h = jnp.where(h > 0, h, jnp.exp(jnp.minimum(h, 0.0)) - 1.0)
    out = jnp.dot(h, w2t_ref[...], preferred_element_type=jnp.float32)
    out_ref[...] = out + b2_ref[...]


def _mlp(x, W1, b1, W2, b2):
    n = x.shape[0]
    return pl.pallas_call(
        _mlp_body,
        out_shape=jax.ShapeDtypeStruct((n, OUT_DIM), jnp.float32),
    )(x, W1.T, b1[None, :], W2.T, b2[None, :])


def _rsqrt16(d):
    i = plsc.bitcast(d, jnp.int32)
    i = jnp.int32(0x5F3759DF) - lax.shift_right_logical(i, 1)
    y = plsc.bitcast(i, jnp.float32)
    for _ in range(3):
        y = y * (1.5 - 0.5 * d * y * y)
    return y


def _norm_body(src_hbm, dst_hbm, w_hbm, norm_hbm,
               srcv, dstv, wv, dinv_v, deg_sh):
    c = lax.axis_index("c")
    s = lax.axis_index("s")

    @pl.when(c == 0)
    def _():
        npt = N_PAD // NS

        @pl.loop(0, npt // 16)
        def _(i):
            dinv_v[pl.ds(i * 16, 16)] = jnp.zeros((16,), jnp.float32)

        pltpu.sync_copy(dinv_v.at[pl.ds(0, npt)],
                        deg_sh.at[pl.ds(s * npt, npt)])
        pltpu.sync_copy(src_hbm.at[s], srcv)
        pltpu.sync_copy(dst_hbm.at[s], dstv)
        pltpu.sync_copy(w_hbm.at[s], wv)
        plsc.subcore_barrier()

        @pl.loop(0, CHUNKS_PT)
        def _(j):
            pltpu.sync_copy(wv.at[j], deg_sh.at[dstv.at[j]], add=True)

        plsc.subcore_barrier()

        pltpu.sync_copy(deg_sh, dinv_v)

        @pl.loop(0, N_PAD // 16)
        def _(i):
            d = dinv_v[pl.ds(i * 16, 16)]
            y = _rsqrt16(jnp.maximum(d, 1e-12))
            dinv_v[pl.ds(i * 16, 16)] = jnp.where(d > 0, y, 0.0)

        @pl.loop(0, CHUNKS_PT)
        def _(j):
            @pl.loop(0, CHUNK // 16)
            def _(e):
                sl = pl.ds(e * 16, 16)
                a = plsc.load_gather(dinv_v, [srcv[j, sl]])
                b = plsc.load_gather(dinv_v, [dstv[j, sl]])
                wv[j, sl] = a * wv[j, sl] * b

        pltpu.sync_copy(wv, norm_hbm.at[s])


def _edge_norm(src3, dst3, w3):
    kfn = pl.kernel(
        _norm_body,
        out_type=jax.ShapeDtypeStruct((NS, CHUNKS_PT, CHUNK), jnp.float32),
        mesh=_mesh,
        scratch_types=[
            pltpu.VMEM((CHUNKS_PT, CHUNK), jnp.int32),
            pltpu.VMEM((CHUNKS_PT, CHUNK), jnp.int32),
            pltpu.VMEM((CHUNKS_PT, CHUNK), jnp.float32),
            pltpu.VMEM((N_PAD,), jnp.float32),
            pltpu.VMEM_SHARED((N_PAD,), jnp.float32),
        ],
        compiler_params=_sc_params,
    )
    return kfn(src3, dst3, w3)


def _prop_body(x_hbm, meta_hbm, agg_hbm, metav, rows0, agg_sh):
    c = lax.axis_index("c")
    s = lax.axis_index("s")
    w = c * NS + s

    @pl.loop(0, UPD)
    def _(r):
        for f in range(OUT_DIM // 16):
            rows0[r, pl.ds(f * 16, 16)] = jnp.zeros((16,), jnp.float32)

    for k in range(NODES_PT // UPD):
        pltpu.sync_copy(rows0,
                        agg_sh.at[pl.ds(s * NODES_PT + k * UPD, UPD)])
    plsc.subcore_barrier()

    @pl.loop(0, CHUNKS_PT2)
    def _(j):
        pltpu.sync_copy(meta_hbm.at[w * CHUNKS_PT2 + j], metav)
        pltpu.sync_copy(x_hbm.at[metav.at[0]], rows0)

        @pl.loop(0, CHUNK // 16)
        def _(g):
            nv = plsc.bitcast(metav[2, pl.ds(g * 16, 16)], jnp.float32)
            for i in range(16):
                n = nv[i]
                e = g * 16 + i
                for f in range(OUT_DIM // 16):
                    sl = pl.ds(f * 16, 16)
                    rows0[e, sl] = rows0[e, sl] * n

        pltpu.sync_copy(rows0, agg_sh.at[metav.at[1]], add=True)

    plsc.subcore_barrier()

    for k in range(NODES_PT // UPD):
        nsl = pl.ds(s * NODES_PT + k * UPD, UPD)
        pltpu.sync_copy(agg_sh.at[nsl], rows0)
        pltpu.sync_copy(rows0, agg_hbm.at[c].at[nsl])


def _propagate(x, meta):
    kfn = pl.kernel(
        _prop_body,
        out_type=jax.ShapeDtypeStruct((NC, N_PAD, OUT_DIM), jnp.float32),
        mesh=_mesh,
        scratch_types=[
            pltpu.VMEM((3, CHUNK), jnp.int32),
            pltpu.VMEM((CHUNK, OUT_DIM), jnp.float32),
            pltpu.VMEM_SHARED((N_PAD, OUT_DIM), jnp.float32),
        ],
        compiler_params=_sc_params,
    )
    return kfn(x, meta)


def _update_body(a0_ref, a1_ref, h_ref, out_ref):
    out_ref[...] = (1.0 - ALPHA) * (a0_ref[...] + a1_ref[...]) \
        + ALPHA * h_ref[...]


def _update(agg2, h):
    return pl.pallas_call(
        _update_body,
        out_shape=jax.ShapeDtypeStruct((N_PAD, OUT_DIM), jnp.float32),
    )(agg2[0], agg2[1], h)


def kernel(x, edge_index, edge_attr, W1, b1, W2, b2):
    src = edge_index[0]
    dst = edge_index[1]
    loop = jnp.arange(N_NODES, dtype=src.dtype)
    pad = E_PAD - (src.shape[0] + N_NODES)
    zpad_i = jnp.zeros((pad,), src.dtype)
    zpad_f = jnp.zeros((pad,), jnp.float32)
    src_flat = jnp.concatenate([src, loop, zpad_i])
    dst_flat = jnp.concatenate([dst, loop, zpad_i])
    w_flat = jnp.concatenate(
        [edge_attr, jnp.ones((N_NODES,), jnp.float32), zpad_f])
    src3 = src_flat.reshape(NS, CHUNKS_PT, CHUNK)
    dst3 = dst_flat.reshape(NS, CHUNKS_PT, CHUNK)
    w3 = w_flat.reshape(NS, CHUNKS_PT, CHUNK)

    norm3 = _edge_norm(src3, dst3, w3)
    norm_bits = lax.bitcast_convert_type(norm3.reshape(-1), jnp.int32)
    nchunks = E_PAD // CHUNK
    meta = jnp.stack(
        [src_flat.reshape(nchunks, CHUNK),
         dst_flat.reshape(nchunks, CHUNK),
         norm_bits.reshape(nchunks, CHUNK)], axis=1)

    h0 = _mlp(x, W1, b1, W2, b2)
    h0 = jnp.concatenate(
        [h0, jnp.zeros((N_PAD - N_NODES, OUT_DIM), jnp.float32)])

    xcur = h0
    for _ in range(APPNP_K):
        xcur = _update(_propagate(xcur, meta), h0)
    h1 = xcur
    for _ in range(APPNP_K):
        xcur = _update(_propagate(xcur, meta), h1)

    return xcur[:N_NODES]
